# baseline (speedup 1.0000x reference)
import functools

import jax
import jax.numpy as jnp
from jax import lax
from jax.experimental import pallas as pl
from jax.experimental.pallas import tpu as pltpu
from jax.experimental.pallas import tpu_sc as plsc

_EPS = 1e-08
_B, _N, _M = 4, 2304, 2304
_NW = 32
_QPW = _B * _N // _NW
_L = 16


def _sc_body(sx_hbm, sy_hbm, tx_hbm, ty_hbm, om_hbm, oa_hbm,
             qx_v, qy_v, tx_v, ty_v, om_v, oa_v):
    c = lax.axis_index("c")
    s = lax.axis_index("s")
    wid = s * 2 + c
    qbase = wid * _QPW
    tbase = (wid // 8) * _M
    pltpu.sync_copy(sx_hbm.at[pl.ds(qbase, _QPW)], qx_v)
    pltpu.sync_copy(sy_hbm.at[pl.ds(qbase, _QPW)], qy_v)
    pltpu.sync_copy(tx_hbm.at[pl.ds(tbase, _M)], tx_v)
    pltpu.sync_copy(ty_hbm.at[pl.ds(tbase, _M)], ty_v)

    for g in range(_QPW // _L):
        qx = qx_v[pl.ds(g * _L, _L)]
        qy = qy_v[pl.ds(g * _L, _L)]

        def body(j, carry, qx=qx, qy=qy):
            m, am = carry
            tvx = tx_v[pl.ds(j * _L, _L)]
            tvy = ty_v[pl.ds(j * _L, _L)]
            tb = j * _L
            for u in range(_L):
                dx = jnp.abs(qx - tvx[u]) + _EPS
                dy = jnp.abs(qy - tvy[u]) + _EPS
                sq = dx * dx + dy * dy
                ltm = sq < m
                am = jnp.where(ltm, tb + u, am)
                m = jnp.minimum(m, sq)
            return m, am

        m0 = jnp.full((_L,), jnp.inf, jnp.float32)
        am0 = jnp.zeros((_L,), jnp.int32)
        m, am = lax.fori_loop(0, _M // _L, body, (m0, am0))
        om_v[pl.ds(g * _L, _L)] = m
        oa_v[pl.ds(g * _L, _L)] = am

    pltpu.sync_copy(om_v, om_hbm.at[pl.ds(qbase, _QPW)])
    pltpu.sync_copy(oa_v, oa_hbm.at[pl.ds(qbase, _QPW)])


def _pairwise_min_sc(sxf, syf, txf, tyf):
    mesh = plsc.VectorSubcoreMesh(core_axis_name="c", subcore_axis_name="s")
    run = functools.partial(
        pl.kernel,
        mesh=mesh,
        out_type=[
            jax.ShapeDtypeStruct((_B * _N,), jnp.float32),
            jax.ShapeDtypeStruct((_B * _N,), jnp.int32),
        ],
        scratch_types=[
            pltpu.VMEM((_QPW,), jnp.float32),
            pltpu.VMEM((_QPW,), jnp.float32),
            pltpu.VMEM((_M,), jnp.float32),
            pltpu.VMEM((_M,), jnp.float32),
            pltpu.VMEM((_QPW,), jnp.float32),
            pltpu.VMEM((_QPW,), jnp.int32),
        ],
    )(_sc_body)
    return run(sxf, syf, txf, tyf)


def _tc_body(sx_ref, sy_ref, tx_ref, ty_ref, omin_ref, oarg_ref, *, tn, m):
    sx = sx_ref[0]
    sy = sy_ref[0]
    tx = tx_ref[0]
    ty = ty_ref[0]
    dx = jnp.abs(sx - tx) + _EPS
    dy = jnp.abs(sy - ty) + _EPS
    s = dx * dx + dy * dy
    mn = jnp.min(s, axis=1, keepdims=True)
    idx = jax.lax.broadcasted_iota(jnp.int32, (tn, m), 1)
    am = jnp.min(jnp.where(s <= mn, idx, m), axis=1, keepdims=True)
    omin_ref[0] = mn
    oarg_ref[0] = am


def _pairwise_min_tc(sx, sy, tx, ty, *, tn=384, interpret=False):
    b, n, _ = sx.shape
    m = tx.shape[2]
    grid = (b, n // tn)
    src_spec = pl.BlockSpec((1, tn, 1), lambda bi, i: (bi, i, 0))
    tgt_spec = pl.BlockSpec((1, 1, m), lambda bi, i: (bi, 0, 0))
    out_spec = pl.BlockSpec((1, tn, 1), lambda bi, i: (bi, i, 0))
    mn, am = pl.pallas_call(
        functools.partial(_tc_body, tn=tn, m=m),
        grid=grid,
        in_specs=[src_spec, src_spec, tgt_spec, tgt_spec],
        out_specs=[out_spec, out_spec],
        out_shape=[
            jax.ShapeDtypeStruct((b, n, 1), jnp.float32),
            jax.ShapeDtypeStruct((b, n, 1), jnp.int32),
        ],
        interpret=interpret,
    )(sx, sy, tx, ty)
    return mn, am


@jax.jit
def kernel(source_uv_warped, target_uv_pred):
    b = source_uv_warped.shape[0]
    src = jnp.reshape(source_uv_warped, (b, -1, 2))
    tgt = jnp.reshape(target_uv_pred, (b, -1, 2))
    n = src.shape[1]
    sxf = src[:, :, 0].reshape(-1)
    syf = src[:, :, 1].reshape(-1)
    txf = tgt[:, :, 0].reshape(-1)
    tyf = tgt[:, :, 1].reshape(-1)
    mn, am = _pairwise_min_sc(sxf, syf, txf, tyf)
    return (jnp.sqrt(mn.reshape(b, n)), am.reshape(b, n))


# speedup vs baseline: 2.0253x; 2.0253x over previous
import functools

import jax
import jax.numpy as jnp
from jax import lax
from jax.experimental import pallas as pl
from jax.experimental.pallas import tpu as pltpu
from jax.experimental.pallas import tpu_sc as plsc

_EPS = 1e-08
_B, _N, _M = 4, 2304, 2304
_NW = 32
_QPW = _B * _N // _NW
_L = 16


def _sc_body(sx_hbm, sy_hbm, tx_hbm, ty_hbm, om_hbm, oa_hbm,
             qx_v, qy_v, tx_v, ty_v, om_v, oa_v, *, k):
    chunk = k // 8
    c = lax.axis_index("c")
    s = lax.axis_index("s")
    wid = s * 2 + c
    b = wid // 8
    cw = wid % 8
    qbase = b * _N + cw * chunk
    obase = b * k + cw * chunk
    tbase = b * _M
    pltpu.sync_copy(sx_hbm.at[pl.ds(qbase, chunk)], qx_v)
    pltpu.sync_copy(sy_hbm.at[pl.ds(qbase, chunk)], qy_v)
    pltpu.sync_copy(tx_hbm.at[pl.ds(tbase, _M)], tx_v)
    pltpu.sync_copy(ty_hbm.at[pl.ds(tbase, _M)], ty_v)

    for g in range(chunk // _L):
        qx = qx_v[pl.ds(g * _L, _L)]
        qy = qy_v[pl.ds(g * _L, _L)]

        def body(j, carry, qx=qx, qy=qy):
            m, am = carry
            tvx = tx_v[pl.ds(j * _L, _L)]
            tvy = ty_v[pl.ds(j * _L, _L)]
            tb = j * _L
            for u in range(_L):
                dx = jnp.abs(qx - tvx[u]) + _EPS
                dy = jnp.abs(qy - tvy[u]) + _EPS
                sq = dx * dx + dy * dy
                ltm = sq < m
                am = jnp.where(ltm, tb + u, am)
                m = jnp.minimum(m, sq)
            return m, am

        m0 = jnp.full((_L,), jnp.inf, jnp.float32)
        am0 = jnp.zeros((_L,), jnp.int32)
        m, am = lax.fori_loop(0, _M // _L, body, (m0, am0))
        om_v[pl.ds(g * _L, _L)] = m
        oa_v[pl.ds(g * _L, _L)] = am

    pltpu.sync_copy(om_v, om_hbm.at[pl.ds(obase, chunk)])
    pltpu.sync_copy(oa_v, oa_hbm.at[pl.ds(obase, chunk)])


def _pairwise_min_sc(sxf, syf, txf, tyf, *, k=_N):
    chunk = k // 8
    mesh = plsc.VectorSubcoreMesh(core_axis_name="c", subcore_axis_name="s")
    run = functools.partial(
        pl.kernel,
        mesh=mesh,
        out_type=[
            jax.ShapeDtypeStruct((_B * k,), jnp.float32),
            jax.ShapeDtypeStruct((_B * k,), jnp.int32),
        ],
        scratch_types=[
            pltpu.VMEM((chunk,), jnp.float32),
            pltpu.VMEM((chunk,), jnp.float32),
            pltpu.VMEM((_M,), jnp.float32),
            pltpu.VMEM((_M,), jnp.float32),
            pltpu.VMEM((chunk,), jnp.float32),
            pltpu.VMEM((chunk,), jnp.int32),
        ],
    )(functools.partial(_sc_body, k=k))
    return run(sxf, syf, txf, tyf)


def _tc_body(sx_ref, sy_ref, tx_ref, ty_ref, omin_ref, oarg_ref, *, tn, m):
    sx = sx_ref[0]
    sy = sy_ref[0]
    tx = tx_ref[0]
    ty = ty_ref[0]
    dx = jnp.abs(sx - tx) + _EPS
    dy = jnp.abs(sy - ty) + _EPS
    s = dx * dx + dy * dy
    mn = jnp.min(s, axis=1, keepdims=True)
    idx = jax.lax.broadcasted_iota(jnp.int32, (tn, m), 1)
    am = jnp.min(jnp.where(s <= mn, idx, m), axis=1, keepdims=True)
    omin_ref[0] = mn
    oarg_ref[0] = am


def _pairwise_min_tc(sx, sy, tx, ty, *, tn=384, interpret=False):
    b, n, _ = sx.shape
    m = tx.shape[2]
    grid = (b, n // tn)
    src_spec = pl.BlockSpec((1, tn, 1), lambda bi, i: (bi, i, 0))
    tgt_spec = pl.BlockSpec((1, 1, m), lambda bi, i: (bi, 0, 0))
    out_spec = pl.BlockSpec((1, tn, 1), lambda bi, i: (bi, i, 0))
    mn, am = pl.pallas_call(
        functools.partial(_tc_body, tn=tn, m=m),
        grid=grid,
        in_specs=[src_spec, src_spec, tgt_spec, tgt_spec],
        out_specs=[out_spec, out_spec],
        out_shape=[
            jax.ShapeDtypeStruct((b, n, 1), jnp.float32),
            jax.ShapeDtypeStruct((b, n, 1), jnp.int32),
        ],
        interpret=interpret,
    )(sx, sy, tx, ty)
    return mn, am


_K_SC = 768


@jax.jit
def kernel(source_uv_warped, target_uv_pred):
    b = source_uv_warped.shape[0]
    src = jnp.reshape(source_uv_warped, (b, -1, 2))
    tgt = jnp.reshape(target_uv_pred, (b, -1, 2))
    n = src.shape[1]
    sxf = src[:, :, 0].reshape(-1)
    syf = src[:, :, 1].reshape(-1)
    txf = tgt[:, :, 0].reshape(-1)
    tyf = tgt[:, :, 1].reshape(-1)
    mn_sc, am_sc = _pairwise_min_sc(sxf, syf, txf, tyf, k=_K_SC)
    sx = src[:, _K_SC:, 0:1]
    sy = src[:, _K_SC:, 1:2]
    tx = tgt[:, :, 0][:, None, :]
    ty = tgt[:, :, 1][:, None, :]
    mn_tc, am_tc = _pairwise_min_tc(sx, sy, tx, ty, tn=384)
    nk = n - _K_SC
    mn = jnp.concatenate(
        [mn_sc.reshape(b, _K_SC), mn_tc.reshape(b, nk)], axis=1)
    am = jnp.concatenate(
        [am_sc.reshape(b, _K_SC), am_tc.reshape(b, nk)], axis=1)
    return (jnp.sqrt(mn), am)
